# baseline (speedup 1.0000x reference)
import functools

import numpy as np

import jax
import jax.numpy as jnp
from jax.experimental import pallas as pl
from jax.experimental.pallas import tpu as pltpu

_B = 1024
_T = 10
_K = 4
_NSLOT = sum(k + 1 for k in range(1, _K + 1))


def _slot_base(k):
    return (k - 1) * (k + 2) // 2


def _build_schedule():
    steps = []
    for r in range(_T):
        for c in range(_T):
            steps.append((r, c, 0))
    for r in range(_T):
        for c in range(r + _K + 1, _T):
            steps.append((r, c, 1))
    n = len(steps)

    last = {}
    for r in range(_T):
        end_strip = min(r + _K, _T - 1)
        last[r] = end_strip * _T + (_T - 1)
    for t, (r, c, p2) in enumerate(steps):
        if p2:
            last[r] = max(last[r], t)

    order = sorted(range(_T), key=lambda r: (last[r], r))
    copy_step = {}
    prev = -1
    for r in order:
        s = max(last[r], prev + 1)
        assert s < n
        copy_step[r] = s
        prev = s

    out_idx = np.zeros(n, np.int32)
    cpy = np.zeros(n, np.int32)
    t0 = 0
    for r in order:
        out_idx[t0 : copy_step[r] + 1] = r
        cpy[copy_step[r]] = 1
        t0 = copy_step[r] + 1
    assert t0 == n

    r_arr = np.array([s[0] for s in steps], np.int32)
    c_arr = np.array([s[1] for s in steps], np.int32)
    p2_arr = np.array([s[2] for s in steps], np.int32)
    return n, r_arr, c_arr, p2_arr, out_idx, cpy


_NSTEPS, _R_ARR, _C_ARR, _P2_ARR, _OUT_IDX, _CPY = _build_schedule()


def _s1_body(x_ref, w1_ref, o_ref):
    o_ref[...] = jnp.dot(
        x_ref[...].astype(jnp.bfloat16),
        w1_ref[...].astype(jnp.bfloat16),
        preferred_element_type=jnp.float32,
    ).astype(jnp.bfloat16)


def _gcn_body(
    n_valid,
    r_ref, c_ref, p2_ref, oi_ref, cp_ref,
    adj_ref, s1_ref, b1_ref, w2_ref, b2_ref,
    out_ref,
    h_acc, g_all, out_acc, diag, held,
):
    t = pl.program_id(0)
    r = r_ref[t]
    c = c_ref[t]
    p2 = p2_ref[t]

    a = adj_ref[...].astype(jnp.bfloat16)
    cols = jax.lax.broadcasted_iota(jnp.int32, a.shape, 1)
    a = jnp.where(cols < n_valid - c * a.shape[1], a, 0)

    @pl.when(t == 0)
    def _init():
        out_acc[...] = jnp.zeros_like(out_acc)

    @pl.when(p2 == 0)
    def _phase1():
        part = jnp.dot(a, s1_ref[...], preferred_element_type=jnp.float32)

        @pl.when(c == 0)
        def _():
            h_acc[...] = part

        @pl.when(c != 0)
        def _():
            h_acc[...] += part

        @pl.when(c < r)
        def _():
            g_c = g_all[pl.ds(c * _B, _B), :]
            out_acc[pl.ds(r * _B, _B), :] += jnp.dot(
                a, g_c, preferred_element_type=jnp.float32
            )

        @pl.when(c == r)
        def _():
            diag[...] = a

        @pl.when((c > r) & (c <= r + _K))
        def _():
            k = c - r
            base = (k - 1) * (k + 2) // 2
            slot = base + jax.lax.rem(r, k + 1)
            held[slot] = a

        @pl.when(c == _T - 1)
        def _strip_end():
            h = jnp.maximum(h_acc[...] + b1_ref[...], 0.0)
            g_r = jnp.dot(
                h.astype(jnp.bfloat16), w2_ref[...], preferred_element_type=jnp.float32
            )
            rows = jax.lax.broadcasted_iota(jnp.int32, g_r.shape, 0) + r * _B
            g_r = jnp.where(rows < n_valid, g_r, 0.0).astype(jnp.bfloat16)
            g_all[pl.ds(r * _B, _B), :] = g_r
            out_acc[pl.ds(r * _B, _B), :] += jnp.dot(
                diag[...], g_r, preferred_element_type=jnp.float32
            )
            for kk in range(1, _K + 1):
                @pl.when(r >= kk)
                def _(kk=kk):
                    r2 = r - kk
                    slot = _slot_base(kk) + jax.lax.rem(r2, kk + 1)
                    out_acc[pl.ds(r2 * _B, _B), :] += jnp.dot(
                        held[slot], g_r, preferred_element_type=jnp.float32
                    )

    @pl.when(p2 == 1)
    def _phase2():
        g_c = g_all[pl.ds(c * _B, _B), :]
        out_acc[pl.ds(r * _B, _B), :] += jnp.dot(
            a, g_c, preferred_element_type=jnp.float32
        )

    @pl.when(cp_ref[t] == 1)
    def _copy_out():
        orow = oi_ref[t]
        out_ref[...] = out_acc[pl.ds(orow * _B, _B), :] + b2_ref[...]


def kernel(x, adj, W1, b1, W2, b2):
    n, nfeat = x.shape
    nhid = W1.shape[1]
    nout = W2.shape[1]
    npad = _T * _B
    b1r = b1.reshape(1, nhid)
    b2r = b2.reshape(1, nout)

    xp = jnp.pad(x, ((0, npad - n), (0, 0)))
    s1p = pl.pallas_call(
        _s1_body,
        out_shape=jax.ShapeDtypeStruct((npad, nhid), jnp.bfloat16),
    )(xp, W1)

    grid_spec = pltpu.PrefetchScalarGridSpec(
        num_scalar_prefetch=5,
        grid=(_NSTEPS,),
        in_specs=[
            pl.BlockSpec((_B, _B), lambda t, rr, cc, pp, oo, kk: (rr[t], cc[t])),
            pl.BlockSpec((_B, nhid), lambda t, rr, cc, pp, oo, kk: (cc[t], 0)),
            pl.BlockSpec((1, nhid), lambda t, rr, cc, pp, oo, kk: (0, 0)),
            pl.BlockSpec((nhid, nout), lambda t, rr, cc, pp, oo, kk: (0, 0)),
            pl.BlockSpec((1, nout), lambda t, rr, cc, pp, oo, kk: (0, 0)),
        ],
        out_specs=pl.BlockSpec((_B, nout), lambda t, rr, cc, pp, oo, kk: (oo[t], 0)),
        scratch_shapes=[
            pltpu.VMEM((_B, nhid), jnp.float32),
            pltpu.VMEM((npad, nout), jnp.bfloat16),
            pltpu.VMEM((npad, nout), jnp.float32),
            pltpu.VMEM((_B, _B), jnp.bfloat16),
            pltpu.VMEM((_NSLOT, _B, _B), jnp.bfloat16),
        ],
    )

    out = pl.pallas_call(
        functools.partial(_gcn_body, n),
        grid_spec=grid_spec,
        out_shape=jax.ShapeDtypeStruct((n, nout), jnp.float32),
        compiler_params=pltpu.CompilerParams(
            dimension_semantics=("arbitrary",),
        ),
    )(
        jnp.asarray(_R_ARR),
        jnp.asarray(_C_ARR),
        jnp.asarray(_P2_ARR),
        jnp.asarray(_OUT_IDX),
        jnp.asarray(_CPY),
        adj,
        s1p,
        b1r,
        W2.astype(jnp.bfloat16),
        b2r,
    )
    return out


# speedup vs baseline: 1.1260x; 1.1260x over previous
import functools

import numpy as np

import jax
import jax.numpy as jnp
from jax.experimental import pallas as pl
from jax.experimental.pallas import tpu as pltpu

_B = 1280
_T = 8
_K = 2
_NSLOT = sum(k + 1 for k in range(1, _K + 1))


def _slot_base(k):
    return (k - 1) * (k + 2) // 2


def _build_schedule():
    steps = []
    for r in range(_T):
        for c in range(_T):
            steps.append((r, c, 0))
    for r in range(_T):
        for c in range(r + _K + 1, _T):
            steps.append((r, c, 1))
    n = len(steps)

    last = {}
    for r in range(_T):
        end_strip = min(r + _K, _T - 1)
        last[r] = end_strip * _T + (_T - 1)
    for t, (r, c, p2) in enumerate(steps):
        if p2:
            last[r] = max(last[r], t)

    order = sorted(range(_T), key=lambda r: (last[r], r))
    copy_step = {}
    prev = -1
    for r in order:
        s = max(last[r], prev + 1)
        assert s < n
        copy_step[r] = s
        prev = s

    out_idx = np.zeros(n, np.int32)
    cpy = np.zeros(n, np.int32)
    t0 = 0
    for r in order:
        out_idx[t0 : copy_step[r] + 1] = r
        cpy[copy_step[r]] = 1
        t0 = copy_step[r] + 1
    assert t0 == n

    r_arr = np.array([s[0] for s in steps], np.int32)
    c_arr = np.array([s[1] for s in steps], np.int32)
    p2_arr = np.array([s[2] for s in steps], np.int32)
    return n, r_arr, c_arr, p2_arr, out_idx, cpy


_NSTEPS, _R_ARR, _C_ARR, _P2_ARR, _OUT_IDX, _CPY = _build_schedule()


def _s1_body(x_ref, w1_ref, o_ref):
    o_ref[...] = jnp.dot(
        x_ref[...].astype(jnp.bfloat16),
        w1_ref[...].astype(jnp.bfloat16),
        preferred_element_type=jnp.float32,
    ).astype(jnp.bfloat16)


def _gcn_body(
    n_valid,
    nf,
    r_ref, c_ref, p2_ref, oi_ref, cp_ref,
    adj_ref, s1_ref, b1_ref, w2_ref, b2_ref,
    out_ref,
    h_acc, s1g, out_acc, diag, held,
):
    t = pl.program_id(0)
    r = r_ref[t]
    c = c_ref[t]
    p2 = p2_ref[t]

    @pl.when(t == 0)
    def _init():
        s1g[...] = jnp.zeros_like(s1g)
        out_acc[...] = jnp.zeros_like(out_acc)

    @pl.when((p2 == 0) & (r == 0))
    def _fill_s1():
        s1g[pl.ds(c * _B, _B), 0:nf] = s1_ref[...]

    def use_block(a):
        rhs = s1g[pl.ds(c * _B, _B), :]
        res = jnp.dot(a, rhs, preferred_element_type=jnp.float32)

        @pl.when(p2 == 0)
        def _layer1():
            part = res[:, 0:nf]

            @pl.when(c == 0)
            def _():
                h_acc[...] = part

            @pl.when(c != 0)
            def _():
                h_acc[...] += part

            @pl.when(c == r)
            def _():
                diag[...] = a

            if _K > 0:
                @pl.when((c > r) & (c <= r + _K))
                def _():
                    k = c - r
                    base = (k - 1) * (k + 2) // 2
                    slot = base + jax.lax.rem(r, k + 1)
                    held[slot] = a

        @pl.when((c < r) | (p2 == 1))
        def _layer2():
            out_acc[pl.ds(r * _B, _B), :] += res[:, nf:]

    a_raw = adj_ref[...].astype(jnp.bfloat16)

    @pl.when(c != _T - 1)
    def _interior():
        use_block(a_raw)

    @pl.when(c == _T - 1)
    def _edge():
        lane = jax.lax.broadcasted_iota(jnp.int32, (1, _B), 1)
        use_block(jnp.where(lane < (n_valid - c * _B), a_raw, 0))

    @pl.when((p2 == 0) & (c == _T - 1))
    def _strip_end():
        h = jnp.maximum(h_acc[...] + b1_ref[...], 0.0)
        g_r = jnp.dot(
            h.astype(jnp.bfloat16), w2_ref[...], preferred_element_type=jnp.float32
        )
        rows = jax.lax.broadcasted_iota(jnp.int32, g_r.shape, 0) + r * _B
        g_r = jnp.where(rows < n_valid, g_r, 0.0).astype(jnp.bfloat16)
        s1g[pl.ds(r * _B, _B), nf:] = g_r
        out_acc[pl.ds(r * _B, _B), :] += jnp.dot(
            diag[...], g_r, preferred_element_type=jnp.float32
        )
        for kk in range(1, _K + 1):
            @pl.when(r >= kk)
            def _(kk=kk):
                r2 = r - kk
                slot = _slot_base(kk) + jax.lax.rem(r2, kk + 1)
                out_acc[pl.ds(r2 * _B, _B), :] += jnp.dot(
                    held[slot], g_r, preferred_element_type=jnp.float32
                )

    @pl.when(cp_ref[t] == 1)
    def _copy_out():
        orow = oi_ref[t]
        out_ref[...] = out_acc[pl.ds(orow * _B, _B), :] + b2_ref[...]


def kernel(x, adj, W1, b1, W2, b2):
    n, nfeat = x.shape
    nhid = W1.shape[1]
    nout = W2.shape[1]
    npad = _T * _B
    b1r = b1.reshape(1, nhid)
    b2r = b2.reshape(1, nout)

    xp = jnp.pad(x, ((0, npad - n), (0, 0)))
    s1p = pl.pallas_call(
        _s1_body,
        out_shape=jax.ShapeDtypeStruct((npad, nhid), jnp.bfloat16),
    )(xp, W1)

    held_shape = (_NSLOT, _B, _B) if _K > 0 else (1, 8, 128)
    grid_spec = pltpu.PrefetchScalarGridSpec(
        num_scalar_prefetch=5,
        grid=(_NSTEPS,),
        in_specs=[
            pl.BlockSpec((_B, _B), lambda t, rr, cc, pp, oo, kk: (rr[t], cc[t])),
            pl.BlockSpec((_B, nhid), lambda t, rr, cc, pp, oo, kk: (cc[t], 0)),
            pl.BlockSpec((1, nhid), lambda t, rr, cc, pp, oo, kk: (0, 0)),
            pl.BlockSpec((nhid, nout), lambda t, rr, cc, pp, oo, kk: (0, 0)),
            pl.BlockSpec((1, nout), lambda t, rr, cc, pp, oo, kk: (0, 0)),
        ],
        out_specs=pl.BlockSpec((_B, nout), lambda t, rr, cc, pp, oo, kk: (oo[t], 0)),
        scratch_shapes=[
            pltpu.VMEM((_B, nhid), jnp.float32),
            pltpu.VMEM((npad, nhid + nout), jnp.bfloat16),
            pltpu.VMEM((npad, nout), jnp.float32),
            pltpu.VMEM((_B, _B), jnp.bfloat16),
            pltpu.VMEM(held_shape, jnp.bfloat16),
        ],
    )

    out = pl.pallas_call(
        functools.partial(_gcn_body, n, nhid),
        grid_spec=grid_spec,
        out_shape=jax.ShapeDtypeStruct((n, nout), jnp.float32),
        compiler_params=pltpu.CompilerParams(
            dimension_semantics=("arbitrary",),
        ),
    )(
        jnp.asarray(_R_ARR),
        jnp.asarray(_C_ARR),
        jnp.asarray(_P2_ARR),
        jnp.asarray(_OUT_IDX),
        jnp.asarray(_CPY),
        adj,
        s1p,
        b1r,
        W2.astype(jnp.bfloat16),
        b2r,
    )
    return out
